# baseline (speedup 1.0000x reference)
import jax
import jax.numpy as jnp
from jax.experimental import pallas as pl


def _loss_body(x_ref, t_ref, o_ref):
    x = x_ref[:]
    y = t_ref[:].astype(jnp.float32)
    C = x.shape[1]

    bce = jnp.sum(jnp.maximum(x, 0.0) - x * y
                  + jnp.log1p(jnp.exp(-jnp.abs(x))))

    s = jax.nn.sigmoid(x)
    es = jnp.exp(s)
    a = jnp.sum(es * (1.0 - y), axis=1)
    p = jnp.sum(y / es, axis=1)
    n_o = jnp.sum(y, axis=1)
    n_z = C - n_o
    den = n_o * n_z
    per = jnp.where(den > 0.0, (a * p) / jnp.maximum(den, 1.0), 0.0)
    corr = jnp.sum(per)

    col = jax.lax.broadcasted_iota(jnp.int32, (1, 128), 1)
    o_ref[:] = (jnp.where(col == 0, bce, 0.0)
                + jnp.where(col == 1, corr, 0.0))


def kernel(logits, targets):
    B, C = logits.shape
    out = pl.pallas_call(
        _loss_body,
        out_shape=jax.ShapeDtypeStruct((1, 128), jnp.float32),
    )(logits, targets)
    bce_mean = out[0, 0] / (B * C)
    corr_mean = out[0, 1] / B
    return 0.8 * bce_mean + 0.2 * corr_mean
